# Initial kernel scaffold; baseline (speedup 1.0000x reference)
import jax
import jax.numpy as jnp
from jax.experimental import pallas as pl


def kernel(input_features, W, b, q_in, q_out, q_ker):
    raise NotImplementedError("write your pallas kernel here")


# speedup vs baseline: 4.2215x; 4.2215x over previous
import functools

import jax
import jax.numpy as jnp
import numpy as np
from jax.experimental import pallas as pl

_THR = 0.15
_BLOCK_O = 256


def _conv_body(x_ref, w_ref, b_ref, qin_bf_ref, qout_ref, qker_ref, out_ref):
    qo = qout_ref[...]
    qk = qker_ref[...]
    qin_bf = qin_bf_ref[...]
    x = x_ref[...]

    ox, oy, oz, ow = qo[:, 0:1], qo[:, 1:2], qo[:, 2:3], qo[:, 3:4]
    cos_half = jnp.float32(np.cos(_THR / 2.0))
    num_k = qk.shape[0]

    acc = jnp.zeros(out_ref.shape, jnp.float32)
    for k in range(num_k):
        kx, ky, kz, kw = qk[k, 0], qk[k, 1], qk[k, 2], qk[k, 3]
        ix = ow * kx + ox * kw + oy * kz - oz * ky
        iy = ow * ky - ox * kz + oy * kw + oz * kx
        iz = ow * kz + ox * ky - oy * kx + oz * kw
        iw = ow * kw - ox * kx - oy * ky - oz * kz
        qik = jnp.concatenate([ix, iy, iz, iw], axis=1)
        dots = jnp.dot(qik.astype(jnp.bfloat16), qin_bf,
                       preferred_element_type=jnp.float32)
        mask = (jnp.abs(dots) > cos_half).astype(jnp.float32)
        counts = jnp.maximum(jnp.sum(mask, axis=1, keepdims=True), 1.0)
        agg = jnp.dot(mask, x, preferred_element_type=jnp.float32) / counts
        acc = acc + jnp.dot(agg, w_ref[k], preferred_element_type=jnp.float32)
    out_ref[...] = acc + b_ref[...]


@functools.partial(jax.jit, static_argnames=())
def kernel(input_features, W, b, q_in, q_out, q_ker):
    n_out = q_out.shape[0]
    n_in, c = input_features.shape
    k, _, d = W.shape
    block_o = min(_BLOCK_O, n_out)
    grid = (n_out // block_o,)
    qin_bf = q_in.T.astype(jnp.bfloat16)

    return pl.pallas_call(
        _conv_body,
        grid=grid,
        in_specs=[
            pl.BlockSpec((n_in, c), lambda i: (0, 0)),
            pl.BlockSpec((k, c, d), lambda i: (0, 0, 0)),
            pl.BlockSpec((1, d), lambda i: (0, 0)),
            pl.BlockSpec((4, n_in), lambda i: (0, 0)),
            pl.BlockSpec((block_o, 4), lambda i: (i, 0)),
            pl.BlockSpec((k, 4), lambda i: (0, 0)),
        ],
        out_specs=pl.BlockSpec((block_o, d), lambda i: (i, 0)),
        out_shape=jax.ShapeDtypeStruct((n_out, d), jnp.float32),
    )(input_features, W, b.reshape(1, d), qin_bf, q_out, q_ker)
